# Initial kernel scaffold; baseline (speedup 1.0000x reference)
import jax
import jax.numpy as jnp
from jax.experimental import pallas as pl


def kernel(x, edge_index, Wf, bf, Wc0, bc0, Wl0, bl0, g0, be0, Wc1, bc1, Wl1, bl1, g1, be1, Wc2, bc2, Wl2, bl2, g2, be2, Wp, bp):
    raise NotImplementedError("write your pallas kernel here")


# speedup vs baseline: 6.2537x; 6.2537x over previous
import functools

import jax
import jax.numpy as jnp
from jax import lax
from jax.experimental import pallas as pl
from jax.experimental.pallas import tpu as pltpu
from jax.experimental.pallas import tpu_sc as plsc

_N = 10000
_D = 128
_E = 320000
_NC = 2
_NS = 16
_NW = _NC * _NS
_EPT = 10240
_EPAD = _EPT * _NW
_IDX_ROWS = _EPAD // 128
_RPW = _EPT // 128
_GRP = 2
_NBLK = _RPW // _GRP
_ACC_ROWS = 10240
_RPT = _ACC_ROWS // _NS


def _deg_body(dst_hbm, ones_hbm, zeros_hbm, out_hbm, dst_v, ones_v, acc):
    cid = lax.axis_index("c")
    sid = lax.axis_index("s")
    wid = sid * _NC + cid

    pltpu.sync_copy(ones_hbm, ones_v)
    r0 = sid * _RPT
    pltpu.sync_copy(zeros_hbm, acc.at[pl.ds(r0, _RPT)])
    plsc.subcore_barrier()

    row_base = wid * _RPW

    def body(b, carry):
        blk = row_base + b * _GRP
        pltpu.sync_copy(dst_hbm.at[pl.ds(blk, _GRP)], dst_v)
        for j in range(_GRP):
            pltpu.sync_copy(ones_v, acc.at[dst_v.at[j]], add=True)
        return carry

    lax.fori_loop(0, _NBLK, body, 0)
    plsc.subcore_barrier()
    pltpu.sync_copy(acc.at[pl.ds(r0, _RPT)], out_hbm.at[cid, pl.ds(r0, _RPT)])


def _mp_body(hp_hbm, src_hbm, dst_hbm, zeros_hbm, out_hbm,
             src_v, dst_v, rows_v, acc, sem):
    cid = lax.axis_index("c")
    sid = lax.axis_index("s")
    wid = sid * _NC + cid

    r0 = sid * _RPT
    pltpu.sync_copy(zeros_hbm, acc.at[pl.ds(r0, _RPT)])
    plsc.subcore_barrier()

    row_base = wid * _RPW

    def body(b, carry):
        blk = row_base + b * _GRP
        pltpu.sync_copy(src_hbm.at[pl.ds(blk, _GRP)], src_v)
        pltpu.sync_copy(dst_hbm.at[pl.ds(blk, _GRP)], dst_v)
        descs = [pltpu.async_copy(hp_hbm.at[src_v.at[j]], rows_v.at[j], sem)
                 for j in range(_GRP)]
        for d in descs:
            d.wait()
        for j in range(_GRP):
            pltpu.sync_copy(rows_v.at[j], acc.at[dst_v.at[j]], add=True)
        return carry

    lax.fori_loop(0, _NBLK, body, 0)
    plsc.subcore_barrier()
    pltpu.sync_copy(acc.at[pl.ds(r0, _RPT)], out_hbm.at[cid, pl.ds(r0, _RPT)])


@functools.lru_cache(maxsize=None)
def _sc_kernels():
    mesh = plsc.VectorSubcoreMesh(core_axis_name="c", subcore_axis_name="s",
                                  num_cores=_NC, num_subcores=_NS)
    deg_kernel = pl.kernel(
        _deg_body,
        out_type=jax.ShapeDtypeStruct((_NC, _ACC_ROWS, _D), jnp.float32),
        mesh=mesh,
        scratch_types=[
            pltpu.VMEM((_GRP, 128), jnp.int32),
            pltpu.VMEM((128, _D), jnp.float32),
            pltpu.VMEM_SHARED((_ACC_ROWS, _D), jnp.float32),
        ],
    )
    mp_kernel = pl.kernel(
        _mp_body,
        out_type=jax.ShapeDtypeStruct((_NC, _ACC_ROWS, _D), jnp.float32),
        mesh=mesh,
        scratch_types=[
            pltpu.VMEM((_GRP, 128), jnp.int32),
            pltpu.VMEM((_GRP, 128), jnp.int32),
            pltpu.VMEM((_GRP, 128, _D), jnp.float32),
            pltpu.VMEM_SHARED((_ACC_ROWS, _D), jnp.float32),
            pltpu.SemaphoreType.DMA,
        ],
    )
    return deg_kernel, mp_kernel


_BLK = 1000
_GRID = _N // _BLK


def _dinv_blk(degp):
    deg = degp[0, :, 0:1] + degp[1, :, 0:1] + 1.0
    return lax.rsqrt(deg)


def _pre_body(x_ref, w_ref, degp_ref, hp_ref):
    dinv = _dinv_blk(degp_ref[...])
    hp_ref[...] = jnp.dot(x_ref[...], w_ref[...],
                          preferred_element_type=jnp.float32) * dinv


def _comb0_body(m_ref, hp_ref, degp_ref, w_ref, b_ref, x1_ref, hp1_ref):
    dinv = _dinv_blk(degp_ref[...])
    m = m_ref[...]
    x1 = (m[0] + m[1] + hp_ref[...]) * dinv + b_ref[...]
    x1_ref[...] = x1
    hp1_ref[...] = jnp.dot(x1, w_ref[...],
                           preferred_element_type=jnp.float32) * dinv


def _comb_mid_body(m_ref, hp_ref, xp_ref, degp_ref, bc_ref, wl_ref, bl_ref,
                   gs_ref, be_ref, wn_ref, xn_ref, hpn_ref):
    dinv = _dinv_blk(degp_ref[...])
    m = m_ref[...]
    g = (m[0] + m[1] + hp_ref[...]) * dinv + bc_ref[...]
    y = g + jnp.dot(xp_ref[...], wl_ref[...],
                    preferred_element_type=jnp.float32) + bl_ref[...]
    xn = jnp.maximum(y * gs_ref[...] + be_ref[...], 0.0)
    xn_ref[...] = xn
    hpn_ref[...] = jnp.dot(xn, wn_ref[...],
                           preferred_element_type=jnp.float32) * dinv


def _comb_last_body(m_ref, hp_ref, xp_ref, degp_ref, bc_ref, wl_ref, bl_ref,
                    gs_ref, be_ref, wp_ref, bp_ref, out_ref):
    dinv = _dinv_blk(degp_ref[...])
    m = m_ref[...]
    g = (m[0] + m[1] + hp_ref[...]) * dinv + bc_ref[...]
    y = g + jnp.dot(xp_ref[...], wl_ref[...],
                    preferred_element_type=jnp.float32) + bl_ref[...]
    xn = jnp.maximum(y * gs_ref[...] + be_ref[...], 0.0)
    out_ref[...] = jnp.dot(xn, wp_ref[...],
                           preferred_element_type=jnp.float32) + bp_ref[...]


def _nd(shape):
    ndim = len(shape)
    if ndim == 2:
        return pl.BlockSpec((_BLK, shape[1]), lambda i: (i, 0))
    return pl.BlockSpec((shape[0], _BLK, shape[2]), lambda i: (0, i, 0))


def _full(shape):
    ndim = len(shape)
    return pl.BlockSpec(shape, (lambda i: (0,) * ndim))


def _tc_call(body, in_arrays, blocked, out_specs_blocked, out_shapes):
    in_specs = [_nd(a.shape) if blk else _full(a.shape)
                for a, blk in zip(in_arrays, blocked)]
    out_specs = [_nd(s.shape) if blk else _full(s.shape)
                 for s, blk in zip(out_shapes, out_specs_blocked)]
    single = len(out_shapes) == 1
    return pl.pallas_call(
        body,
        grid=(_GRID,),
        in_specs=in_specs,
        out_specs=out_specs[0] if single else out_specs,
        out_shape=out_shapes[0] if single else out_shapes,
        compiler_params=pltpu.CompilerParams(
            dimension_semantics=("parallel",)),
    )(*in_arrays)


def kernel(x, edge_index, Wf, bf, Wc0, bc0, Wl0, bl0, g0, be0,
           Wc1, bc1, Wl1, bl1, g1, be1, Wc2, bc2, Wl2, bl2, g2, be2, Wp, bp):
    f32 = jnp.float32
    eps = 1e-5
    bn_scale = 1.0 / jnp.sqrt(jnp.float32(1.0 + eps))

    src = edge_index[0]
    dst = edge_index[1]
    pad = _EPAD - _E
    src_p = jnp.concatenate(
        [src, jnp.zeros((pad,), jnp.int32)]).reshape(_IDX_ROWS, 128)
    dst_p = jnp.concatenate(
        [dst, jnp.full((pad,), _N, jnp.int32)]).reshape(_IDX_ROWS, 128)

    zeros_d = jnp.zeros((_RPT, _D), f32)
    ones_d = jnp.ones((128, _D), f32)

    nd2 = jax.ShapeDtypeStruct((_N, _D), f32)

    deg_kernel, mp_kernel = _sc_kernels()

    degp = deg_kernel(dst_p, ones_d, zeros_d)

    def mp(hp):
        return mp_kernel(hp, src_p, dst_p, zeros_d)

    def row(v):
        return v.reshape(1, _D).astype(f32)

    hp0 = _tc_call(_pre_body, [x, Wf, degp], [True, False, True],
                   [True], [nd2])
    m0 = mp(hp0)
    x1, hp1 = _tc_call(_comb0_body, [m0, hp0, degp, Wc0, row(bf)],
                       [True, True, True, False, False],
                       [True, True], [nd2, nd2])

    m1 = mp(hp1)
    x2, hp2 = _tc_call(
        _comb_mid_body,
        [m1, hp1, x1, degp, row(bc0), Wl0, row(bl0),
         row(g0) * bn_scale, row(be0), Wc1],
        [True, True, True, True, False, False, False, False, False, False],
        [True, True], [nd2, nd2])

    m2 = mp(hp2)
    x3, hp3 = _tc_call(
        _comb_mid_body,
        [m2, hp2, x2, degp, row(bc1), Wl1, row(bl1),
         row(g1) * bn_scale, row(be1), Wc2],
        [True, True, True, True, False, False, False, False, False, False],
        [True, True], [nd2, nd2])

    m3 = mp(hp3)
    out = _tc_call(
        _comb_last_body,
        [m3, hp3, x3, degp, row(bc2), Wl2, row(bl2),
         row(g2) * bn_scale, row(be2), Wp, row(bp)],
        [True, True, True, True, False, False, False, False, False, False,
         False],
        [True], [jax.ShapeDtypeStruct((_N, _D), f32)])
    return out


# speedup vs baseline: 6.9338x; 1.1087x over previous
import functools

import jax
import jax.numpy as jnp
from jax import lax
from jax.experimental import pallas as pl
from jax.experimental.pallas import tpu as pltpu
from jax.experimental.pallas import tpu_sc as plsc

_N = 10000
_D = 128
_E = 320000
_NC = 2
_NS = 16
_NW = _NC * _NS
_EPT = 10240
_EPAD = _EPT * _NW
_IDX_ROWS = _EPAD // 128
_RPW = _EPT // 128
_STAGE = 16
_NOUT = _RPW // _STAGE
_ACC_ROWS = 10240
_RPT = _ACC_ROWS // _NS


def _deg_body(dst_hbm, ones_hbm, zeros_hbm, out_hbm, dst_v, ones_v, acc, sem):
    cid = lax.axis_index("c")
    sid = lax.axis_index("s")
    wid = sid * _NC + cid

    pltpu.sync_copy(ones_hbm, ones_v)
    r0 = sid * _RPT
    pltpu.sync_copy(zeros_hbm, acc.at[pl.ds(r0, _RPT)])
    plsc.subcore_barrier()

    row_base = wid * _RPW

    def body(c, carry):
        blk = row_base + c * _STAGE
        pltpu.sync_copy(dst_hbm.at[pl.ds(blk, _STAGE)], dst_v)
        descs = [pltpu.async_copy(ones_v, acc.at[dst_v.at[j]], sem, add=True)
                 for j in range(_STAGE)]
        for d in descs:
            d.wait()
        return carry

    lax.fori_loop(0, _NOUT, body, 0)
    plsc.subcore_barrier()
    pltpu.sync_copy(acc.at[pl.ds(r0, _RPT)], out_hbm.at[cid, pl.ds(r0, _RPT)])


def _mp_body(hp_hbm, src_hbm, dst_hbm, zeros_hbm, out_hbm,
             src_v, dst_v, rows_v, acc, sem_g, sem_s0, sem_s1):
    cid = lax.axis_index("c")
    sid = lax.axis_index("s")
    wid = sid * _NC + cid

    r0 = sid * _RPT
    pltpu.sync_copy(zeros_hbm, acc.at[pl.ds(r0, _RPT)])
    plsc.subcore_barrier()

    row_base = wid * _RPW

    def body(c, carry):
        blk = row_base + c * _STAGE
        pltpu.sync_copy(src_hbm.at[pl.ds(blk, _STAGE)], src_v)
        pltpu.sync_copy(dst_hbm.at[pl.ds(blk, _STAGE)], dst_v)
        gathers = [None, None]
        scatters = [None, None]
        sem_s = [sem_s0, sem_s1]
        gathers[0] = pltpu.async_copy(
            hp_hbm.at[src_v.at[0]], rows_v.at[0], sem_g)
        for k in range(_STAGE):
            b = k % 2
            gathers[b].wait()
            if k >= 1:
                scatters[1 - b].wait()
            if k + 1 < _STAGE:
                gathers[1 - b] = pltpu.async_copy(
                    hp_hbm.at[src_v.at[k + 1]], rows_v.at[1 - b], sem_g)
            scatters[b] = pltpu.async_copy(
                rows_v.at[b], acc.at[dst_v.at[k]], sem_s[b], add=True)
        scatters[(_STAGE - 1) % 2].wait()
        return carry

    lax.fori_loop(0, _NOUT, body, 0)
    plsc.subcore_barrier()
    pltpu.sync_copy(acc.at[pl.ds(r0, _RPT)], out_hbm.at[cid, pl.ds(r0, _RPT)])


@functools.lru_cache(maxsize=None)
def _sc_kernels():
    mesh = plsc.VectorSubcoreMesh(core_axis_name="c", subcore_axis_name="s",
                                  num_cores=_NC, num_subcores=_NS)
    deg_kernel = pl.kernel(
        _deg_body,
        out_type=jax.ShapeDtypeStruct((_NC, _ACC_ROWS, _D), jnp.float32),
        mesh=mesh,
        scratch_types=[
            pltpu.VMEM((_STAGE, 128), jnp.int32),
            pltpu.VMEM((128, _D), jnp.float32),
            pltpu.VMEM_SHARED((_ACC_ROWS, _D), jnp.float32),
            pltpu.SemaphoreType.DMA,
        ],
    )
    mp_kernel = pl.kernel(
        _mp_body,
        out_type=jax.ShapeDtypeStruct((_NC, _ACC_ROWS, _D), jnp.float32),
        mesh=mesh,
        scratch_types=[
            pltpu.VMEM((_STAGE, 128), jnp.int32),
            pltpu.VMEM((_STAGE, 128), jnp.int32),
            pltpu.VMEM((2, 128, _D), jnp.float32),
            pltpu.VMEM_SHARED((_ACC_ROWS, _D), jnp.float32),
            pltpu.SemaphoreType.DMA,
            pltpu.SemaphoreType.DMA,
            pltpu.SemaphoreType.DMA,
        ],
    )
    return deg_kernel, mp_kernel


_BLK = 1000
_GRID = _N // _BLK


def _dinv_blk(degp):
    deg = degp[0, :, 0:1] + degp[1, :, 0:1] + 1.0
    return lax.rsqrt(deg)


def _pre_body(x_ref, w_ref, degp_ref, hp_ref):
    dinv = _dinv_blk(degp_ref[...])
    hp_ref[...] = jnp.dot(x_ref[...], w_ref[...],
                          preferred_element_type=jnp.float32) * dinv


def _comb0_body(m_ref, hp_ref, degp_ref, w_ref, b_ref, x1_ref, hp1_ref):
    dinv = _dinv_blk(degp_ref[...])
    m = m_ref[...]
    x1 = (m[0] + m[1] + hp_ref[...]) * dinv + b_ref[...]
    x1_ref[...] = x1
    hp1_ref[...] = jnp.dot(x1, w_ref[...],
                           preferred_element_type=jnp.float32) * dinv


def _comb_mid_body(m_ref, hp_ref, xp_ref, degp_ref, bc_ref, wl_ref, bl_ref,
                   gs_ref, be_ref, wn_ref, xn_ref, hpn_ref):
    dinv = _dinv_blk(degp_ref[...])
    m = m_ref[...]
    g = (m[0] + m[1] + hp_ref[...]) * dinv + bc_ref[...]
    y = g + jnp.dot(xp_ref[...], wl_ref[...],
                    preferred_element_type=jnp.float32) + bl_ref[...]
    xn = jnp.maximum(y * gs_ref[...] + be_ref[...], 0.0)
    xn_ref[...] = xn
    hpn_ref[...] = jnp.dot(xn, wn_ref[...],
                           preferred_element_type=jnp.float32) * dinv


def _comb_last_body(m_ref, hp_ref, xp_ref, degp_ref, bc_ref, wl_ref, bl_ref,
                    gs_ref, be_ref, wp_ref, bp_ref, out_ref):
    dinv = _dinv_blk(degp_ref[...])
    m = m_ref[...]
    g = (m[0] + m[1] + hp_ref[...]) * dinv + bc_ref[...]
    y = g + jnp.dot(xp_ref[...], wl_ref[...],
                    preferred_element_type=jnp.float32) + bl_ref[...]
    xn = jnp.maximum(y * gs_ref[...] + be_ref[...], 0.0)
    out_ref[...] = jnp.dot(xn, wp_ref[...],
                           preferred_element_type=jnp.float32) + bp_ref[...]


def _nd(shape):
    ndim = len(shape)
    if ndim == 2:
        return pl.BlockSpec((_BLK, shape[1]), lambda i: (i, 0))
    return pl.BlockSpec((shape[0], _BLK, shape[2]), lambda i: (0, i, 0))


def _full(shape):
    ndim = len(shape)
    return pl.BlockSpec(shape, (lambda i: (0,) * ndim))


def _tc_call(body, in_arrays, blocked, out_specs_blocked, out_shapes):
    in_specs = [_nd(a.shape) if blk else _full(a.shape)
                for a, blk in zip(in_arrays, blocked)]
    out_specs = [_nd(s.shape) if blk else _full(s.shape)
                 for s, blk in zip(out_shapes, out_specs_blocked)]
    single = len(out_shapes) == 1
    return pl.pallas_call(
        body,
        grid=(_GRID,),
        in_specs=in_specs,
        out_specs=out_specs[0] if single else out_specs,
        out_shape=out_shapes[0] if single else out_shapes,
        compiler_params=pltpu.CompilerParams(
            dimension_semantics=("parallel",)),
    )(*in_arrays)


def kernel(x, edge_index, Wf, bf, Wc0, bc0, Wl0, bl0, g0, be0,
           Wc1, bc1, Wl1, bl1, g1, be1, Wc2, bc2, Wl2, bl2, g2, be2, Wp, bp):
    f32 = jnp.float32
    eps = 1e-5
    bn_scale = 1.0 / jnp.sqrt(jnp.float32(1.0 + eps))

    src = edge_index[0]
    dst = edge_index[1]
    pad = _EPAD - _E
    src_p = jnp.concatenate(
        [src, jnp.zeros((pad,), jnp.int32)]).reshape(_IDX_ROWS, 128)
    dst_p = jnp.concatenate(
        [dst, jnp.full((pad,), _N, jnp.int32)]).reshape(_IDX_ROWS, 128)

    zeros_d = jnp.zeros((_RPT, _D), f32)
    ones_d = jnp.ones((128, _D), f32)

    nd2 = jax.ShapeDtypeStruct((_N, _D), f32)

    deg_kernel, mp_kernel = _sc_kernels()

    degp = deg_kernel(dst_p, ones_d, zeros_d)

    def mp(hp):
        return mp_kernel(hp, src_p, dst_p, zeros_d)

    def row(v):
        return v.reshape(1, _D).astype(f32)

    hp0 = _tc_call(_pre_body, [x, Wf, degp], [True, False, True],
                   [True], [nd2])
    m0 = mp(hp0)
    x1, hp1 = _tc_call(_comb0_body, [m0, hp0, degp, Wc0, row(bf)],
                       [True, True, True, False, False],
                       [True, True], [nd2, nd2])

    m1 = mp(hp1)
    x2, hp2 = _tc_call(
        _comb_mid_body,
        [m1, hp1, x1, degp, row(bc0), Wl0, row(bl0),
         row(g0) * bn_scale, row(be0), Wc1],
        [True, True, True, True, False, False, False, False, False, False],
        [True, True], [nd2, nd2])

    m2 = mp(hp2)
    x3, hp3 = _tc_call(
        _comb_mid_body,
        [m2, hp2, x2, degp, row(bc1), Wl1, row(bl1),
         row(g1) * bn_scale, row(be1), Wc2],
        [True, True, True, True, False, False, False, False, False, False],
        [True, True], [nd2, nd2])

    m3 = mp(hp3)
    out = _tc_call(
        _comb_last_body,
        [m3, hp3, x3, degp, row(bc2), Wl2, row(bl2),
         row(g2) * bn_scale, row(be2), Wp, row(bp)],
        [True, True, True, True, False, False, False, False, False, False,
         False],
        [True], [jax.ShapeDtypeStruct((_N, _D), f32)])
    return out
